# baseline (speedup 1.0000x reference)
import functools

import jax
import jax.numpy as jnp
from jax import lax
from jax.experimental import pallas as pl
from jax.experimental.pallas import tpu as pltpu
from jax.experimental.pallas import tpu_sc as plsc

_T = 0.0025
_M = 1.0
_K = 16
_N = 1024
_BIG = 1.0e9

_NC = 2
_NS = 16
_NW = _NC * _NS
_RPW = _N // _NW
_L = 16
_NCHAIN = 4
_CPC = _N // _L // _NCHAIN
_NSEL = _RPW * _L


def _prep_kernel(yi_ref, yit_ref, gm_ref, c_ref, e2_ref):
    yi = yi_ref[...]
    yit = yit_ref[...]
    yi_n = yi / jnp.sqrt(jnp.sum(yi * yi, axis=1, keepdims=True) + 1e-12)
    yit_n = yit / jnp.sqrt(jnp.sum(yit * yit, axis=1, keepdims=True) + 1e-12)

    dims = (((1,), (1,)), ((), ()))
    g = jax.lax.dot_general(yi_n, yi_n, dims, preferred_element_type=jnp.float32)
    c = jax.lax.dot_general(yit_n, yi_n, dims, preferred_element_type=jnp.float32)

    row_iota = jax.lax.broadcasted_iota(jnp.int32, (_N, _N), 0)
    col_iota = jax.lax.broadcasted_iota(jnp.int32, (_N, _N), 1)
    gm = jnp.where(row_iota == col_iota, _BIG, -g)
    gm_ref[...] = gm
    c_ref[...] = c

    gmin = jnp.min(gm, axis=1)
    d1 = 0.5 * jnp.sqrt(jnp.maximum(2.0 + 2.0 * gmin, 0.0) + 1e-12)
    diff = yi_n - yit_n
    dd = 0.5 * jnp.sqrt(jnp.sum(diff * diff, axis=1) + 1e-12)
    out2 = dd + _M - d1
    e2 = jnp.sum(jnp.where(out2 > 0, out2, 0.0))
    e2_ref[...] = jnp.reshape(e2, (1, 1))


def _half_sqrt(x):
    i = plsc.bitcast(x, jnp.int32)
    i = 0x5F3759DF - jax.lax.shift_right_logical(i, 1)
    r = plsc.bitcast(i, jnp.float32)
    for _ in range(3):
        r = r * (1.5 - 0.5 * x * r * r)
    return 0.5 * x * r


def _sc_body(gm_hbm, c_hbm, out_hbm, abuf, cbuf, obuf, sem):
    c_id = lax.axis_index("c")
    s_id = lax.axis_index("s")
    wid = s_id * _NC + c_id
    base = wid * _RPW

    _QR = 8
    _NQ = _RPW // _QR
    copies = []
    for q in range(_NQ):
        copies.append((
            pltpu.async_copy(
                gm_hbm.at[pl.ds(base + q * _QR, _QR)],
                abuf.at[pl.ds(q * _QR, _QR)], sem),
            pltpu.async_copy(
                c_hbm.at[pl.ds(base + q * _QR, _QR)],
                cbuf.at[pl.ds(q * _QR, _QR)], sem),
        ))

    lane_iota = jax.lax.iota(jnp.int32, _L)

    def select_row(r):
        bests = []
        bestis = []
        for ch in range(_NCHAIN):
            v = abuf[r, pl.ds(ch * _L, _L)]
            iv = lane_iota + (ch * _L)
            bk, bv = plsc.sort_key_val(v, iv, descending=False)
            bests.append(bk)
            bestis.append(bv)
        for t in range(1, _CPC):
            for ch in range(_NCHAIN):
                off = (t * _NCHAIN + ch) * _L
                v = abuf[r, pl.ds(off, _L)]
                iv = lane_iota + off
                dk, dv = plsc.sort_key_val(v, iv, descending=True)
                take = dk < bests[ch]
                nb = jnp.where(take, dk, bests[ch])
                ni = jnp.where(take, dv, bestis[ch])
                bests[ch], bestis[ch] = plsc.sort_key_val(nb, ni, descending=False)

        def merge(bk0, bi0, bk1, bi1):
            rk = lax.rev(bk1, (0,))
            ri = lax.rev(bi1, (0,))
            take = rk < bk0
            return jnp.where(take, rk, bk0), jnp.where(take, ri, bi0)

        mk0, mi0 = merge(bests[0], bestis[0], bests[1], bestis[1])
        mk0, mi0 = plsc.sort_key_val(mk0, mi0, descending=False)
        mk1, mi1 = merge(bests[2], bestis[2], bests[3], bestis[3])
        mk1, mi1 = plsc.sort_key_val(mk1, mi1, descending=False)
        fk, fi = merge(mk0, mi0, mk1, mi1)

        dsel = _half_sqrt(jnp.maximum(2.0 + 2.0 * fk, 0.0) + 1e-12)
        rvec = jnp.full((_L,), r, dtype=jnp.int32)
        cg = plsc.load_gather(cbuf, [rvec, fi])
        dc = _half_sqrt(jnp.maximum(2.0 - 2.0 * cg, 0.0) + 1e-12)
        df = dsel - dc
        term = df * df - _T
        return jnp.where(term > 0.0, term, 0.0)

    def pair_body(i, vacc):
        return vacc + select_row(2 * i) + select_row(2 * i + 1)

    vacc = jnp.zeros((_L,), jnp.float32)
    for q in range(_NQ):
        copies[q][0].wait()
        copies[q][1].wait()
        vacc = lax.fori_loop(q * _QR // 2, (q + 1) * _QR // 2, pair_body, vacc)
    obuf[...] = vacc
    pltpu.sync_copy(obuf, out_hbm.at[wid])


_sc_select = functools.partial(
    pl.kernel,
    out_type=jax.ShapeDtypeStruct((_NW, _L), jnp.float32),
    mesh=plsc.VectorSubcoreMesh(core_axis_name="c", subcore_axis_name="s"),
    compiler_params=pltpu.CompilerParams(
        needs_layout_passes=False, use_tc_tiling_on_sc=True),
    scratch_types=[
        pltpu.VMEM((_RPW, _N), jnp.float32),
        pltpu.VMEM((_RPW, _N), jnp.float32),
        pltpu.VMEM((_L,), jnp.float32),
        pltpu.SemaphoreType.DMA,
    ],
)(_sc_body)


@jax.jit
def kernel(yi, yi_t):
    gm, c, e2 = pl.pallas_call(
        _prep_kernel,
        out_shape=[
            jax.ShapeDtypeStruct((_N, _N), jnp.float32),
            jax.ShapeDtypeStruct((_N, _N), jnp.float32),
            jax.ShapeDtypeStruct((1, 1), jnp.float32),
        ],
    )(yi, yi_t)
    partials = _sc_select(gm, c)
    e1 = jnp.sum(partials)
    e2s = e2[0, 0]
    return (e1 + e2s, e1, e2s)


# speedup vs baseline: 1.3881x; 1.3881x over previous
import functools

import jax
import jax.numpy as jnp
from jax import lax
from jax.experimental import pallas as pl
from jax.experimental.pallas import tpu as pltpu
from jax.experimental.pallas import tpu_sc as plsc

_T = 0.0025
_M = 1.0
_K = 16
_N = 1024
_BIG = 1.0e9

_NC = 2
_NS = 16
_NW = _NC * _NS
_RPW = _N // _NW
_L = 16
_NCHAIN = 4
_CPC = _N // _L // _NCHAIN
_NSEL = _RPW * _L


def _prep_kernel(yi_ref, yit_ref, gm_ref, c_ref, e2_ref):
    yi = yi_ref[...]
    yit = yit_ref[...]
    yi_n = yi / jnp.sqrt(jnp.sum(yi * yi, axis=1, keepdims=True) + 1e-12)
    yit_n = yit / jnp.sqrt(jnp.sum(yit * yit, axis=1, keepdims=True) + 1e-12)

    dims = (((1,), (1,)), ((), ()))
    g = jax.lax.dot_general(yi_n, yi_n, dims, preferred_element_type=jnp.float32)
    c = jax.lax.dot_general(yit_n, yi_n, dims, preferred_element_type=jnp.float32)

    row_iota = jax.lax.broadcasted_iota(jnp.int32, (_N, _N), 0)
    col_iota = jax.lax.broadcasted_iota(jnp.int32, (_N, _N), 1)
    gm = jnp.where(row_iota == col_iota, _BIG, -g)
    gm_ref[...] = gm
    c_ref[...] = c

    gmin = jnp.min(gm, axis=1)
    d1 = 0.5 * jnp.sqrt(jnp.maximum(2.0 + 2.0 * gmin, 0.0) + 1e-12)
    diff = yi_n - yit_n
    dd = 0.5 * jnp.sqrt(jnp.sum(diff * diff, axis=1) + 1e-12)
    out2 = dd + _M - d1
    e2 = jnp.sum(jnp.where(out2 > 0, out2, 0.0))
    e2_ref[...] = jnp.reshape(e2, (1, 1))


def _half_sqrt(x):
    i = plsc.bitcast(x, jnp.int32)
    i = 0x5F3759DF - jax.lax.shift_right_logical(i, 1)
    r = plsc.bitcast(i, jnp.float32)
    for _ in range(3):
        r = r * (1.5 - 0.5 * x * r * r)
    return 0.5 * x * r


def _sc_body(gm_hbm, c_hbm, out_hbm, abuf, cbuf, obuf, sem):
    c_id = lax.axis_index("c")
    s_id = lax.axis_index("s")
    wid = s_id * _NC + c_id
    base = wid * _RPW

    _QR = 8
    _NQ = _RPW // _QR
    copies = []
    for q in range(_NQ):
        copies.append((
            pltpu.async_copy(
                gm_hbm.at[pl.ds(base + q * _QR, _QR)],
                abuf.at[pl.ds(q * _QR, _QR)], sem),
            pltpu.async_copy(
                c_hbm.at[pl.ds(base + q * _QR, _QR)],
                cbuf.at[pl.ds(q * _QR, _QR)], sem),
        ))

    lane_iota = jax.lax.iota(jnp.int32, _L)

    def select_row(r):
        bests = []
        bestis = []
        for ch in range(_NCHAIN):
            v = abuf[r, pl.ds(ch * _L, _L)]
            iv = lane_iota + (ch * _L)
            bk, bv = plsc.sort_key_val(v, iv, descending=False)
            bests.append(bk)
            bestis.append(bv)
        for t in range(1, _CPC):
            for ch in range(_NCHAIN):
                off = (t * _NCHAIN + ch) * _L
                v = abuf[r, pl.ds(off, _L)]
                iv = lane_iota + off
                dk, dv = plsc.sort_key_val(v, iv, descending=True)
                take = dk < bests[ch]
                nb = jnp.where(take, dk, bests[ch])
                ni = jnp.where(take, dv, bestis[ch])
                bests[ch], bestis[ch] = plsc.sort_key_val(nb, ni, descending=False)

        def merge(bk0, bi0, bk1, bi1):
            rk = lax.rev(bk1, (0,))
            ri = lax.rev(bi1, (0,))
            take = rk < bk0
            return jnp.where(take, rk, bk0), jnp.where(take, ri, bi0)

        mk0, mi0 = merge(bests[0], bestis[0], bests[1], bestis[1])
        mk0, mi0 = plsc.sort_key_val(mk0, mi0, descending=False)
        mk1, mi1 = merge(bests[2], bestis[2], bests[3], bestis[3])
        mk1, mi1 = plsc.sort_key_val(mk1, mi1, descending=False)
        fk, fi = merge(mk0, mi0, mk1, mi1)

        dsel = _half_sqrt(jnp.maximum(2.0 + 2.0 * fk, 0.0) + 1e-12)
        rvec = jnp.full((_L,), r, dtype=jnp.int32)
        cg = plsc.load_gather(cbuf, [rvec, fi])
        dc = _half_sqrt(jnp.maximum(2.0 - 2.0 * cg, 0.0) + 1e-12)
        df = dsel - dc
        term = df * df - _T
        return jnp.where(term > 0.0, term, 0.0)

    def row_body(r, vacc):
        return vacc + select_row(r)

    vacc = jnp.zeros((_L,), jnp.float32)
    for q in range(_NQ):
        copies[q][0].wait()
        copies[q][1].wait()
        vacc = lax.fori_loop(q * _QR, (q + 1) * _QR, row_body, vacc)
    obuf[...] = vacc
    pltpu.sync_copy(obuf, out_hbm.at[wid])


_sc_select = functools.partial(
    pl.kernel,
    out_type=jax.ShapeDtypeStruct((_NW, _L), jnp.float32),
    mesh=plsc.VectorSubcoreMesh(core_axis_name="c", subcore_axis_name="s"),
    compiler_params=pltpu.CompilerParams(
        needs_layout_passes=False, use_tc_tiling_on_sc=True),
    scratch_types=[
        pltpu.VMEM((_RPW, _N), jnp.float32),
        pltpu.VMEM((_RPW, _N), jnp.float32),
        pltpu.VMEM((_L,), jnp.float32),
        pltpu.SemaphoreType.DMA,
    ],
)(_sc_body)


@jax.jit
def kernel(yi, yi_t):
    gm, c, e2 = pl.pallas_call(
        _prep_kernel,
        out_shape=[
            jax.ShapeDtypeStruct((_N, _N), jnp.float32),
            jax.ShapeDtypeStruct((_N, _N), jnp.float32),
            jax.ShapeDtypeStruct((1, 1), jnp.float32),
        ],
    )(yi, yi_t)
    partials = _sc_select(gm, c)
    e1 = jnp.sum(partials)
    e2s = e2[0, 0]
    return (e1 + e2s, e1, e2s)


# speedup vs baseline: 1.4427x; 1.0393x over previous
import functools

import jax
import jax.numpy as jnp
from jax import lax
from jax.experimental import pallas as pl
from jax.experimental.pallas import tpu as pltpu
from jax.experimental.pallas import tpu_sc as plsc

_T = 0.0025
_M = 1.0
_K = 16
_N = 1024
_BIG = 1.0e9

_NC = 2
_NS = 16
_NW = _NC * _NS
_RPW = _N // _NW
_L = 16
_NCHAIN = 4
_CPC = _N // _L // _NCHAIN
_NSEL = _RPW * _L


def _prep_kernel(yi_ref, yit_ref, gm_ref, c_ref, e2_ref):
    yi = yi_ref[...]
    yit = yit_ref[...]
    yi_n = yi / jnp.sqrt(jnp.sum(yi * yi, axis=1, keepdims=True) + 1e-12)
    yit_n = yit / jnp.sqrt(jnp.sum(yit * yit, axis=1, keepdims=True) + 1e-12)

    dims = (((1,), (1,)), ((), ()))
    g = jax.lax.dot_general(yi_n, yi_n, dims, preferred_element_type=jnp.float32)
    c = jax.lax.dot_general(yit_n, yi_n, dims, preferred_element_type=jnp.float32)

    row_iota = jax.lax.broadcasted_iota(jnp.int32, (_N, _N), 0)
    col_iota = jax.lax.broadcasted_iota(jnp.int32, (_N, _N), 1)
    gm = jnp.where(row_iota == col_iota, _BIG, -g)
    gm_ref[...] = gm
    c_ref[...] = c

    gmin = jnp.min(gm, axis=1)
    d1 = 0.5 * jnp.sqrt(jnp.maximum(2.0 + 2.0 * gmin, 0.0) + 1e-12)
    diff = yi_n - yit_n
    dd = 0.5 * jnp.sqrt(jnp.sum(diff * diff, axis=1) + 1e-12)
    out2 = dd + _M - d1
    e2 = jnp.sum(jnp.where(out2 > 0, out2, 0.0))
    e2_ref[...] = jnp.reshape(e2, (1, 1))


def _half_sqrt(x):
    i = plsc.bitcast(x, jnp.int32)
    i = 0x5F3759DF - jax.lax.shift_right_logical(i, 1)
    r = plsc.bitcast(i, jnp.float32)
    for _ in range(3):
        r = r * (1.5 - 0.5 * x * r * r)
    return 0.5 * x * r


def _sc_body(gm_hbm, c_hbm, out_hbm, abuf, cbuf, obuf, sem):
    c_id = lax.axis_index("c")
    s_id = lax.axis_index("s")
    wid = s_id * _NC + c_id
    base = wid * _RPW

    cp_c = pltpu.async_copy(c_hbm.at[pl.ds(base, _RPW)], cbuf, sem)
    pltpu.sync_copy(gm_hbm.at[pl.ds(base, _RPW)], abuf)

    lane_iota = jax.lax.iota(jnp.int32, _L)

    def select_row(r):
        bests = []
        bestis = []
        for ch in range(_NCHAIN):
            v = abuf[r, pl.ds(ch * _L, _L)]
            iv = lane_iota + (ch * _L)
            bk, bv = plsc.sort_key_val(v, iv, descending=False)
            bests.append(bk)
            bestis.append(bv)
        for t in range(1, _CPC):
            for ch in range(_NCHAIN):
                off = (t * _NCHAIN + ch) * _L
                v = abuf[r, pl.ds(off, _L)]
                iv = lane_iota + off
                dk, dv = plsc.sort_key_val(v, iv, descending=True)
                take = dk < bests[ch]
                nb = jnp.where(take, dk, bests[ch])
                ni = jnp.where(take, dv, bestis[ch])
                bests[ch], bestis[ch] = plsc.sort_key_val(nb, ni, descending=False)

        def merge(bk0, bi0, bk1, bi1):
            rk = lax.rev(bk1, (0,))
            ri = lax.rev(bi1, (0,))
            take = rk < bk0
            return jnp.where(take, rk, bk0), jnp.where(take, ri, bi0)

        mk0, mi0 = merge(bests[0], bestis[0], bests[1], bestis[1])
        mk0, mi0 = plsc.sort_key_val(mk0, mi0, descending=False)
        mk1, mi1 = merge(bests[2], bestis[2], bests[3], bestis[3])
        mk1, mi1 = plsc.sort_key_val(mk1, mi1, descending=False)
        fk, fi = merge(mk0, mi0, mk1, mi1)

        dsel = _half_sqrt(jnp.maximum(2.0 + 2.0 * fk, 0.0) + 1e-12)
        rvec = jnp.full((_L,), r, dtype=jnp.int32)
        cg = plsc.load_gather(cbuf, [rvec, fi])
        dc = _half_sqrt(jnp.maximum(2.0 - 2.0 * cg, 0.0) + 1e-12)
        df = dsel - dc
        term = df * df - _T
        return jnp.where(term > 0.0, term, 0.0)

    def row_body(r, vacc):
        return vacc + select_row(r)

    cp_c.wait()
    vacc = lax.fori_loop(0, _RPW, row_body, jnp.zeros((_L,), jnp.float32))
    obuf[...] = vacc
    pltpu.sync_copy(obuf, out_hbm.at[wid])


_sc_select = functools.partial(
    pl.kernel,
    out_type=jax.ShapeDtypeStruct((_NW, _L), jnp.float32),
    mesh=plsc.VectorSubcoreMesh(core_axis_name="c", subcore_axis_name="s"),
    compiler_params=pltpu.CompilerParams(
        needs_layout_passes=False, use_tc_tiling_on_sc=True),
    scratch_types=[
        pltpu.VMEM((_RPW, _N), jnp.float32),
        pltpu.VMEM((_RPW, _N), jnp.float32),
        pltpu.VMEM((_L,), jnp.float32),
        pltpu.SemaphoreType.DMA,
    ],
)(_sc_body)


@jax.jit
def kernel(yi, yi_t):
    gm, c, e2 = pl.pallas_call(
        _prep_kernel,
        out_shape=[
            jax.ShapeDtypeStruct((_N, _N), jnp.float32),
            jax.ShapeDtypeStruct((_N, _N), jnp.float32),
            jax.ShapeDtypeStruct((1, 1), jnp.float32),
        ],
    )(yi, yi_t)
    partials = _sc_select(gm, c)
    e1 = jnp.sum(partials)
    e2s = e2[0, 0]
    return (e1 + e2s, e1, e2s)
